# Initial kernel scaffold; baseline (speedup 1.0000x reference)
import jax
import jax.numpy as jnp
from jax.experimental import pallas as pl


def kernel(x, theta):
    raise NotImplementedError("write your pallas kernel here")


# speedup vs baseline: 2.6697x; 2.6697x over previous
import jax
import jax.numpy as jnp
from jax.experimental import pallas as pl
from jax.experimental.pallas import tpu as pltpu

_I = 3
_J = 700
_ROWS = 16384
_COLS = 1024
_BR = 1024


def _body(theta_ref, x_ref, o_ref):
    t = theta_ref[0]
    c = jnp.cos(2.0 * t)
    s = jnp.sin(2.0 * t)
    blk = x_ref[...]
    xi = blk[:, _I:_I + 1]
    xj = blk[:, _J:_J + 1]
    out0 = xi * c + xj * s
    out1 = xi * s - xj * c
    col = jax.lax.broadcasted_iota(jnp.int32, blk.shape, 1)
    y = jnp.where(col == _I, out0, jnp.where(col == _J, out1, blk))
    o_ref[...] = y


def kernel(x, theta):
    theta_arr = jnp.reshape(theta, (1,)).astype(jnp.float32)
    grid = (_ROWS // _BR,)
    return pl.pallas_call(
        _body,
        grid=grid,
        in_specs=[
            pl.BlockSpec(memory_space=pltpu.SMEM),
            pl.BlockSpec((_BR, _COLS), lambda i: (i, 0)),
        ],
        out_specs=pl.BlockSpec((_BR, _COLS), lambda i: (i, 0)),
        out_shape=jax.ShapeDtypeStruct((_ROWS, _COLS), jnp.float32),
    )(theta_arr, x)


# speedup vs baseline: 2.7567x; 1.0326x over previous
import jax
import jax.numpy as jnp
from jax.experimental import pallas as pl
from jax.experimental.pallas import tpu as pltpu

_I = 3
_J = 700
_ROWS = 16384
_COLS = 1024
_BR = 2048


def _body(theta_ref, x_ref, o_ref):
    t = theta_ref[0]
    c = jnp.cos(2.0 * t)
    s = jnp.sin(2.0 * t)
    blk = x_ref[...]
    xi = blk[:, _I:_I + 1]
    xj = blk[:, _J:_J + 1]
    out0 = xi * c + xj * s
    out1 = xi * s - xj * c
    col = jax.lax.broadcasted_iota(jnp.int32, blk.shape, 1)
    y = jnp.where(col == _I, out0, jnp.where(col == _J, out1, blk))
    o_ref[...] = y


def kernel(x, theta):
    theta_arr = jnp.reshape(theta, (1,)).astype(jnp.float32)
    grid = (_ROWS // _BR,)
    return pl.pallas_call(
        _body,
        grid=grid,
        in_specs=[
            pl.BlockSpec(memory_space=pltpu.SMEM),
            pl.BlockSpec((_BR, _COLS), lambda i: (i, 0)),
        ],
        out_specs=pl.BlockSpec((_BR, _COLS), lambda i: (i, 0)),
        out_shape=jax.ShapeDtypeStruct((_ROWS, _COLS), jnp.float32),
    )(theta_arr, x)


# speedup vs baseline: 2.7615x; 1.0017x over previous
import jax
import jax.numpy as jnp
from jax.experimental import pallas as pl
from jax.experimental.pallas import tpu as pltpu

_I = 3
_J = 700
_ROWS = 16384
_COLS = 1024
_BR = 2048


def _body(theta_ref, x_ref, o_ref):
    t = theta_ref[0]
    c = jnp.cos(2.0 * t)
    s = jnp.sin(2.0 * t)
    o_ref[...] = x_ref[...]
    xi = x_ref[:, _I:_I + 1]
    xj = x_ref[:, _J:_J + 1]
    o_ref[:, _I:_I + 1] = xi * c + xj * s
    o_ref[:, _J:_J + 1] = xi * s - xj * c


def kernel(x, theta):
    theta_arr = jnp.reshape(theta, (1,)).astype(jnp.float32)
    grid = (_ROWS // _BR,)
    return pl.pallas_call(
        _body,
        grid=grid,
        in_specs=[
            pl.BlockSpec(memory_space=pltpu.SMEM),
            pl.BlockSpec((_BR, _COLS), lambda i: (i, 0)),
        ],
        out_specs=pl.BlockSpec((_BR, _COLS), lambda i: (i, 0)),
        out_shape=jax.ShapeDtypeStruct((_ROWS, _COLS), jnp.float32),
    )(theta_arr, x)
